# baseline (speedup 1.0000x reference)
import functools
import jax
import jax.numpy as jnp
from jax import lax
from jax.experimental import pallas as pl
from jax.experimental.pallas import tpu as pltpu
from jax.experimental.pallas import tpu_sc as plsc

N = 10000
E = 320000
D_IN = 128
D_HID = 128
N_CLASS = 40
D2P = 48

NC = 2
NS = 16
NW = NC * NS

EPW = E // NW
CHUNK = 125
NCHUNK = EPW // CHUNK
NP = 10240
ROWS_PT = NP // NS
IDXH = NCHUNK // 2

ROW_BLK = 400


def _mp_kernel(q_hbm, src_hbm, dst_hbm, zeros_hbm, out_hbm,
               idx_s, idx_d, rows0, rows1, agg_sh,
               semg0, semg1, sems0, sems1, semz):
    cid = lax.axis_index("c")
    sid = lax.axis_index("s")
    wid = cid * NS + sid
    base = pl.multiple_of(sid * ROWS_PT, 8)

    def zero_body(t, _):
        pltpu.async_copy(zeros_hbm, agg_sh.at[pl.ds(base + t * 64, 64)], semz)
        return 0

    lax.fori_loop(0, ROWS_PT // 64, zero_body, 0)
    pltpu.sync_copy(src_hbm.at[wid], idx_s)
    pltpu.sync_copy(dst_hbm.at[wid, pl.ds(0, IDXH)], idx_d)

    def zero_wait(t, _):
        pltpu.make_async_copy(zeros_hbm, agg_sh.at[pl.ds(base, 64)],
                              semz).wait()
        return 0

    lax.fori_loop(0, ROWS_PT // 64, zero_wait, 0)
    plsc.subcore_barrier()

    pltpu.async_copy(q_hbm.at[idx_s.at[0]], rows0, semg0)
    pltpu.async_copy(q_hbm.at[idx_s.at[1]], rows1, semg1)

    wide = q_hbm.shape[1] >= 64

    def body(i, _):
        j0 = i * 2
        jl0 = jnp.where(j0 < IDXH, j0, j0 - IDXH)

        @pl.when(j0 == IDXH)
        def _():
            pltpu.sync_copy(dst_hbm.at[wid, pl.ds(IDXH, IDXH)], idx_d)

        pltpu.make_async_copy(q_hbm.at[idx_s.at[j0]], rows0, semg0).wait()
        if wide:
            pltpu.sync_copy(rows0, agg_sh.at[idx_d.at[jl0]], add=True)

            @pl.when(j0 + 2 < NCHUNK)
            def _():
                pltpu.async_copy(q_hbm.at[idx_s.at[j0 + 2]], rows0, semg0)

            pltpu.make_async_copy(q_hbm.at[idx_s.at[j0 + 1]], rows1,
                                  semg1).wait()
            pltpu.sync_copy(rows1, agg_sh.at[idx_d.at[jl0 + 1]], add=True)

            @pl.when(j0 + 3 < NCHUNK)
            def _():
                pltpu.async_copy(q_hbm.at[idx_s.at[j0 + 3]], rows1, semg1)
        else:
            pltpu.async_copy(rows0, agg_sh.at[idx_d.at[jl0]], sems0,
                             add=True)
            pltpu.make_async_copy(q_hbm.at[idx_s.at[j0 + 1]], rows1,
                                  semg1).wait()
            pltpu.async_copy(rows1, agg_sh.at[idx_d.at[jl0 + 1]], sems1,
                             add=True)

            pltpu.make_async_copy(rows0, agg_sh.at[idx_d.at[jl0]],
                                  sems0).wait()

            @pl.when(j0 + 2 < NCHUNK)
            def _():
                pltpu.async_copy(q_hbm.at[idx_s.at[j0 + 2]], rows0, semg0)

            pltpu.make_async_copy(rows1, agg_sh.at[idx_d.at[jl0 + 1]],
                                  sems1).wait()

            @pl.when(j0 + 3 < NCHUNK)
            def _():
                pltpu.async_copy(q_hbm.at[idx_s.at[j0 + 3]], rows1, semg1)

        return 0

    lax.fori_loop(0, NCHUNK // 2, body, 0)

    plsc.subcore_barrier()
    pltpu.sync_copy(agg_sh.at[pl.ds(base, ROWS_PT)],
                    out_hbm.at[cid, pl.ds(base, ROWS_PT)])


def _degree_kernel(src_hbm, dst_hbm, zeros_hbm, out_hbm,
                   idx_s, idx_d, ones_lo, ones_hi, h_sh, semz, semd):
    cid = lax.axis_index("c")
    sid = lax.axis_index("s")
    wid = cid * NS + sid
    base = pl.multiple_of(sid * ROWS_PT, 8)

    lane = lax.iota(jnp.int32, 16)
    lo = jnp.where(lane < 8, 1.0, 0.0).astype(jnp.float32)
    hi = jnp.where(lane < 8, 0.0, 1.0).astype(jnp.float32)

    def fill_body(k, _):
        ones_lo[k] = lo
        ones_hi[k] = hi
        return 0

    lax.fori_loop(0, CHUNK, fill_body, 0)

    def zero_body(t, _):
        pltpu.async_copy(zeros_hbm, h_sh.at[pl.ds(base + t * 64, 64)], semz)
        return 0

    lax.fori_loop(0, ROWS_PT // 64, zero_body, 0)
    pltpu.sync_copy(src_hbm.at[wid, pl.ds(0, IDXH)], idx_s)
    pltpu.sync_copy(dst_hbm.at[wid, pl.ds(0, IDXH)], idx_d)

    def zero_wait(t, _):
        pltpu.make_async_copy(zeros_hbm, h_sh.at[pl.ds(base, 64)], semz).wait()
        return 0

    lax.fori_loop(0, ROWS_PT // 64, zero_wait, 0)
    plsc.subcore_barrier()

    GRP = 4

    for h in range(2):
        if h == 1:
            pltpu.sync_copy(src_hbm.at[wid, pl.ds(IDXH, IDXH)], idx_s)
            pltpu.sync_copy(dst_hbm.at[wid, pl.ds(IDXH, IDXH)], idx_d)

        def grp_body(g, _):
            for kk in range(GRP):
                jl = g * GRP + kk
                pltpu.async_copy(ones_lo, h_sh.at[idx_s.at[jl]], semd,
                                 add=True)
                pltpu.async_copy(ones_hi, h_sh.at[idx_d.at[jl]], semd,
                                 add=True)
            for kk in range(GRP):
                jl = g * GRP + kk
                pltpu.make_async_copy(ones_lo, h_sh.at[idx_s.at[jl]],
                                      semd).wait()
                pltpu.make_async_copy(ones_hi, h_sh.at[idx_d.at[jl]],
                                      semd).wait()
            return 0

        lax.fori_loop(0, IDXH // GRP, grp_body, 0)

    plsc.subcore_barrier()
    pltpu.sync_copy(h_sh.at[pl.ds(base, ROWS_PT)],
                    out_hbm.at[cid, pl.ds(base, ROWS_PT)])


def _degrees(src3, dst3):
    mesh = plsc.VectorSubcoreMesh(core_axis_name="c", subcore_axis_name="s")
    zeros = jnp.zeros((64, 16), jnp.float32)
    return pl.kernel(
        _degree_kernel,
        out_type=jax.ShapeDtypeStruct((NC, NP, 16), jnp.float32),
        mesh=mesh,
        scratch_types=[
            pltpu.VMEM((IDXH, CHUNK), jnp.int32),
            pltpu.VMEM((IDXH, CHUNK), jnp.int32),
            pltpu.VMEM((CHUNK, 16), jnp.float32),
            pltpu.VMEM((CHUNK, 16), jnp.float32),
            pltpu.VMEM_SHARED((NP, 16), jnp.float32),
            pltpu.SemaphoreType.DMA,
            pltpu.SemaphoreType.DMA,
        ],
        compiler_params=pltpu.CompilerParams(use_tc_tiling_on_sc=False),
    )(src3, dst3, zeros)


def _message_pass(q, src3, dst3, d):
    mesh = plsc.VectorSubcoreMesh(core_axis_name="c", subcore_axis_name="s")
    zeros = jnp.zeros((64, d), jnp.float32)
    return pl.kernel(
        _mp_kernel,
        out_type=jax.ShapeDtypeStruct((NC, NP, d), jnp.float32),
        mesh=mesh,
        scratch_types=[
            pltpu.VMEM((NCHUNK, CHUNK), jnp.int32),
            pltpu.VMEM((IDXH, CHUNK), jnp.int32),
            pltpu.VMEM((CHUNK, d), jnp.float32),
            pltpu.VMEM((CHUNK, d), jnp.float32),
            pltpu.VMEM_SHARED((NP, d), jnp.float32),
            pltpu.SemaphoreType.DMA,
            pltpu.SemaphoreType.DMA,
            pltpu.SemaphoreType.DMA,
            pltpu.SemaphoreType.DMA,
            pltpu.SemaphoreType.DMA,
        ],
        compiler_params=None if d % 128 == 0 else pltpu.CompilerParams(
            use_tc_tiling_on_sc=False),
    )(q, src3, dst3, zeros)


def _tc0_body(x_ref, w1_ref, y1_ref):
    y1_ref[...] = jnp.dot(x_ref[...], w1_ref[...],
                          preferred_element_type=jnp.float32)


def _tc0(x, w1):
    grid = (N // ROW_BLK,)
    return pl.pallas_call(
        _tc0_body,
        grid=grid,
        in_specs=[
            pl.BlockSpec((ROW_BLK, D_IN), lambda i: (i, 0)),
            pl.BlockSpec((D_IN, D_HID), lambda i: (0, 0)),
        ],
        out_specs=pl.BlockSpec((ROW_BLK, D_HID), lambda i: (i, 0)),
        out_shape=jax.ShapeDtypeStruct((N, D_HID), jnp.float32),
    )(x, w1)


def _tc1_body(h_ref, y1_ref, q1_ref, dsrc_ref, ddst_ref):
    deg_out = h_ref[0, :, 0:1] + h_ref[1, :, 0:1]
    deg_in = h_ref[0, :, 8:9] + h_ref[1, :, 8:9]
    dinv_s = lax.rsqrt(jnp.maximum(deg_out, 1.0))
    dinv_d = lax.rsqrt(jnp.maximum(deg_in, 1.0))
    q1_ref[...] = y1_ref[...] * dinv_s
    dsrc_ref[...] = dinv_s
    ddst_ref[...] = dinv_d


def _tc1(hist, y1):
    grid = (N // ROW_BLK,)
    return pl.pallas_call(
        _tc1_body,
        grid=grid,
        in_specs=[
            pl.BlockSpec((NC, ROW_BLK, 16), lambda i: (0, i, 0)),
            pl.BlockSpec((ROW_BLK, D_HID), lambda i: (i, 0)),
        ],
        out_specs=[
            pl.BlockSpec((ROW_BLK, D_HID), lambda i: (i, 0)),
            pl.BlockSpec((ROW_BLK, 1), lambda i: (i, 0)),
            pl.BlockSpec((ROW_BLK, 1), lambda i: (i, 0)),
        ],
        out_shape=[
            jax.ShapeDtypeStruct((N, D_HID), jnp.float32),
            jax.ShapeDtypeStruct((N, 1), jnp.float32),
            jax.ShapeDtypeStruct((N, 1), jnp.float32),
        ],
    )(hist, y1)


def _tc2_body(part_ref, ddst_ref, dsrc_ref, b1_ref, w2_ref, q2_ref):
    p = part_ref[0] + part_ref[1]
    h = jnp.maximum(p * ddst_ref[...] + b1_ref[...], 0.0)
    q2_ref[...] = jnp.dot(h * dsrc_ref[...], w2_ref[...],
                          preferred_element_type=jnp.float32)


def _tc2(parts, dinv_dst, dinv_src, b1, w2):
    grid = (N // ROW_BLK,)
    return pl.pallas_call(
        _tc2_body,
        grid=grid,
        in_specs=[
            pl.BlockSpec((NC, ROW_BLK, D_HID), lambda i: (0, i, 0)),
            pl.BlockSpec((ROW_BLK, 1), lambda i: (i, 0)),
            pl.BlockSpec((ROW_BLK, 1), lambda i: (i, 0)),
            pl.BlockSpec((1, D_HID), lambda i: (0, 0)),
            pl.BlockSpec((D_HID, D2P), lambda i: (0, 0)),
        ],
        out_specs=pl.BlockSpec((ROW_BLK, D2P), lambda i: (i, 0)),
        out_shape=jax.ShapeDtypeStruct((N, D2P), jnp.float32),
    )(parts, dinv_dst, dinv_src, b1, w2)


def _tc3_body(part_ref, ddst_ref, b2_ref, out_ref):
    p = part_ref[0, :, :N_CLASS] + part_ref[1, :, :N_CLASS]
    out_ref[...] = p * ddst_ref[...] + b2_ref[...]


def _tc3(parts, dinv_dst, b2):
    grid = (N // ROW_BLK,)
    return pl.pallas_call(
        _tc3_body,
        grid=grid,
        in_specs=[
            pl.BlockSpec((NC, ROW_BLK, D2P), lambda i: (0, i, 0)),
            pl.BlockSpec((ROW_BLK, 1), lambda i: (i, 0)),
            pl.BlockSpec((1, N_CLASS), lambda i: (0, 0)),
        ],
        out_specs=pl.BlockSpec((ROW_BLK, N_CLASS), lambda i: (i, 0)),
        out_shape=jax.ShapeDtypeStruct((N, N_CLASS), jnp.float32),
    )(parts, dinv_dst, b2)


def kernel(x, edge_index, W1, b1, W2, b2):
    src = edge_index[0]
    dst = edge_index[1]
    src3 = src.reshape(NW, NCHUNK, CHUNK)
    dst3 = dst.reshape(NW, NCHUNK, CHUNK)

    hist = _degrees(src3, dst3)
    y1 = _tc0(x, W1)

    q1, dinv_src, dinv_dst = _tc1(hist, y1)
    parts1 = _message_pass(q1, src3, dst3, D_HID)
    w2p = jnp.concatenate([W2, jnp.zeros((D_HID, D2P - N_CLASS), W2.dtype)],
                          axis=1)
    q2 = _tc2(parts1, dinv_dst, dinv_src, b1.reshape(1, D_HID), w2p)
    parts2 = _message_pass(q2, src3, dst3, D2P)
    out = _tc3(parts2, dinv_dst, b2.reshape(1, N_CLASS))
    return out


# speedup vs baseline: 1.0149x; 1.0149x over previous
import functools
import jax
import jax.numpy as jnp
from jax import lax
from jax.experimental import pallas as pl
from jax.experimental.pallas import tpu as pltpu
from jax.experimental.pallas import tpu_sc as plsc

N = 10000
E = 320000
D_IN = 128
D_HID = 128
N_CLASS = 40

NC = 2
NS = 16
NW = NC * NS

EPW = E // NW
CHUNK = 125
NCHUNK = EPW // CHUNK
NP = 10240
ROWS_PT = NP // NS
IDXH = NCHUNK // 2

ROW_BLK = 400


def _mp_kernel(q_hbm, src_hbm, dst_hbm, zeros_hbm, out_hbm,
               idx_s, idx_d, rows0, rows1, agg_sh,
               semg0, semg1, sems0, sems1, semz):
    cid = lax.axis_index("c")
    sid = lax.axis_index("s")
    wid = cid * NS + sid
    base = pl.multiple_of(sid * ROWS_PT, 8)

    def zero_body(t, _):
        pltpu.async_copy(zeros_hbm, agg_sh.at[pl.ds(base + t * 64, 64)], semz)
        return 0

    lax.fori_loop(0, ROWS_PT // 64, zero_body, 0)
    pltpu.sync_copy(src_hbm.at[wid], idx_s)
    pltpu.sync_copy(dst_hbm.at[wid, pl.ds(0, IDXH)], idx_d)

    pltpu.async_copy(q_hbm.at[idx_s.at[0]], rows0, semg0)
    pltpu.async_copy(q_hbm.at[idx_s.at[1]], rows1, semg1)

    def zero_wait(t, _):
        pltpu.make_async_copy(zeros_hbm, agg_sh.at[pl.ds(base, 64)],
                              semz).wait()
        return 0

    lax.fori_loop(0, ROWS_PT // 64, zero_wait, 0)
    plsc.subcore_barrier()

    wide = q_hbm.shape[1] >= 64

    def body(i, _):
        j0 = i * 2
        jl0 = jnp.where(j0 < IDXH, j0, j0 - IDXH)

        @pl.when(j0 == IDXH)
        def _():
            pltpu.sync_copy(dst_hbm.at[wid, pl.ds(IDXH, IDXH)], idx_d)

        pltpu.make_async_copy(q_hbm.at[idx_s.at[j0]], rows0, semg0).wait()
        if wide:
            pltpu.sync_copy(rows0, agg_sh.at[idx_d.at[jl0]], add=True)

            @pl.when(j0 + 2 < NCHUNK)
            def _():
                pltpu.async_copy(q_hbm.at[idx_s.at[j0 + 2]], rows0, semg0)

            pltpu.make_async_copy(q_hbm.at[idx_s.at[j0 + 1]], rows1,
                                  semg1).wait()
            pltpu.sync_copy(rows1, agg_sh.at[idx_d.at[jl0 + 1]], add=True)

            @pl.when(j0 + 3 < NCHUNK)
            def _():
                pltpu.async_copy(q_hbm.at[idx_s.at[j0 + 3]], rows1, semg1)
        else:
            pltpu.async_copy(rows0, agg_sh.at[idx_d.at[jl0]], sems0,
                             add=True)
            pltpu.make_async_copy(q_hbm.at[idx_s.at[j0 + 1]], rows1,
                                  semg1).wait()
            pltpu.async_copy(rows1, agg_sh.at[idx_d.at[jl0 + 1]], sems1,
                             add=True)

            pltpu.make_async_copy(rows0, agg_sh.at[idx_d.at[jl0]],
                                  sems0).wait()

            @pl.when(j0 + 2 < NCHUNK)
            def _():
                pltpu.async_copy(q_hbm.at[idx_s.at[j0 + 2]], rows0, semg0)

            pltpu.make_async_copy(rows1, agg_sh.at[idx_d.at[jl0 + 1]],
                                  sems1).wait()

            @pl.when(j0 + 3 < NCHUNK)
            def _():
                pltpu.async_copy(q_hbm.at[idx_s.at[j0 + 3]], rows1, semg1)

        return 0

    lax.fori_loop(0, NCHUNK // 2, body, 0)

    plsc.subcore_barrier()
    pltpu.sync_copy(agg_sh.at[pl.ds(base, ROWS_PT)],
                    out_hbm.at[cid, pl.ds(base, ROWS_PT)])


def _degree_kernel(src_hbm, dst_hbm, zeros_hbm, out_hbm,
                   idx_s, idx_d, ones_lo, ones_hi, h_sh, semz, semd):
    cid = lax.axis_index("c")
    sid = lax.axis_index("s")
    wid = cid * NS + sid
    base = pl.multiple_of(sid * ROWS_PT, 8)

    lane = lax.iota(jnp.int32, 16)
    lo = jnp.where(lane < 8, 1.0, 0.0).astype(jnp.float32)
    hi = jnp.where(lane < 8, 0.0, 1.0).astype(jnp.float32)

    def fill_body(k, _):
        ones_lo[k] = lo
        ones_hi[k] = hi
        return 0

    lax.fori_loop(0, CHUNK, fill_body, 0)

    def zero_body(t, _):
        pltpu.async_copy(zeros_hbm, h_sh.at[pl.ds(base + t * 64, 64)], semz)
        return 0

    lax.fori_loop(0, ROWS_PT // 64, zero_body, 0)
    pltpu.sync_copy(src_hbm.at[wid, pl.ds(0, IDXH)], idx_s)
    pltpu.sync_copy(dst_hbm.at[wid, pl.ds(0, IDXH)], idx_d)

    def zero_wait(t, _):
        pltpu.make_async_copy(zeros_hbm, h_sh.at[pl.ds(base, 64)], semz).wait()
        return 0

    lax.fori_loop(0, ROWS_PT // 64, zero_wait, 0)
    plsc.subcore_barrier()

    GRP = 4

    for h in range(2):
        if h == 1:
            pltpu.sync_copy(src_hbm.at[wid, pl.ds(IDXH, IDXH)], idx_s)
            pltpu.sync_copy(dst_hbm.at[wid, pl.ds(IDXH, IDXH)], idx_d)

        def grp_body(g, _):
            for kk in range(GRP):
                jl = g * GRP + kk
                pltpu.async_copy(ones_lo, h_sh.at[idx_s.at[jl]], semd,
                                 add=True)
                pltpu.async_copy(ones_hi, h_sh.at[idx_d.at[jl]], semd,
                                 add=True)
            for kk in range(GRP):
                jl = g * GRP + kk
                pltpu.make_async_copy(ones_lo, h_sh.at[idx_s.at[jl]],
                                      semd).wait()
                pltpu.make_async_copy(ones_hi, h_sh.at[idx_d.at[jl]],
                                      semd).wait()
            return 0

        lax.fori_loop(0, IDXH // GRP, grp_body, 0)

    plsc.subcore_barrier()
    pltpu.sync_copy(h_sh.at[pl.ds(base, ROWS_PT)],
                    out_hbm.at[cid, pl.ds(base, ROWS_PT)])


def _degrees(src3, dst3):
    mesh = plsc.VectorSubcoreMesh(core_axis_name="c", subcore_axis_name="s")
    zeros = jnp.zeros((64, 16), jnp.float32)
    return pl.kernel(
        _degree_kernel,
        out_type=jax.ShapeDtypeStruct((NC, NP, 16), jnp.float32),
        mesh=mesh,
        scratch_types=[
            pltpu.VMEM((IDXH, CHUNK), jnp.int32),
            pltpu.VMEM((IDXH, CHUNK), jnp.int32),
            pltpu.VMEM((CHUNK, 16), jnp.float32),
            pltpu.VMEM((CHUNK, 16), jnp.float32),
            pltpu.VMEM_SHARED((NP, 16), jnp.float32),
            pltpu.SemaphoreType.DMA,
            pltpu.SemaphoreType.DMA,
        ],
        compiler_params=pltpu.CompilerParams(use_tc_tiling_on_sc=False),
    )(src3, dst3, zeros)


def _message_pass(q, src3, dst3, d):
    mesh = plsc.VectorSubcoreMesh(core_axis_name="c", subcore_axis_name="s")
    zeros = jnp.zeros((64, d), jnp.float32)
    return pl.kernel(
        _mp_kernel,
        out_type=jax.ShapeDtypeStruct((NC, NP, d), jnp.float32),
        mesh=mesh,
        scratch_types=[
            pltpu.VMEM((NCHUNK, CHUNK), jnp.int32),
            pltpu.VMEM((IDXH, CHUNK), jnp.int32),
            pltpu.VMEM((CHUNK, d), jnp.float32),
            pltpu.VMEM((CHUNK, d), jnp.float32),
            pltpu.VMEM_SHARED((NP, d), jnp.float32),
            pltpu.SemaphoreType.DMA,
            pltpu.SemaphoreType.DMA,
            pltpu.SemaphoreType.DMA,
            pltpu.SemaphoreType.DMA,
            pltpu.SemaphoreType.DMA,
        ],
        compiler_params=None if d % 128 == 0 else pltpu.CompilerParams(
            use_tc_tiling_on_sc=False),
    )(q, src3, dst3, zeros)


def _tc0_body(x_ref, w1_ref, y1_ref):
    y1_ref[...] = jnp.dot(x_ref[...], w1_ref[...],
                          preferred_element_type=jnp.float32)


def _tc0(x, w1):
    grid = (N // ROW_BLK,)
    return pl.pallas_call(
        _tc0_body,
        grid=grid,
        in_specs=[
            pl.BlockSpec((ROW_BLK, D_IN), lambda i: (i, 0)),
            pl.BlockSpec((D_IN, D_HID), lambda i: (0, 0)),
        ],
        out_specs=pl.BlockSpec((ROW_BLK, D_HID), lambda i: (i, 0)),
        out_shape=jax.ShapeDtypeStruct((N, D_HID), jnp.float32),
    )(x, w1)


def _tc1_body(h_ref, y1_ref, q1_ref, dsrc_ref, ddst_ref):
    deg_out = h_ref[0, :, 0:1] + h_ref[1, :, 0:1]
    deg_in = h_ref[0, :, 8:9] + h_ref[1, :, 8:9]
    dinv_s = lax.rsqrt(jnp.maximum(deg_out, 1.0))
    dinv_d = lax.rsqrt(jnp.maximum(deg_in, 1.0))
    q1_ref[...] = y1_ref[...] * dinv_s
    dsrc_ref[...] = dinv_s
    ddst_ref[...] = dinv_d


def _tc1(hist, y1):
    grid = (N // ROW_BLK,)
    return pl.pallas_call(
        _tc1_body,
        grid=grid,
        in_specs=[
            pl.BlockSpec((NC, ROW_BLK, 16), lambda i: (0, i, 0)),
            pl.BlockSpec((ROW_BLK, D_HID), lambda i: (i, 0)),
        ],
        out_specs=[
            pl.BlockSpec((ROW_BLK, D_HID), lambda i: (i, 0)),
            pl.BlockSpec((ROW_BLK, 1), lambda i: (i, 0)),
            pl.BlockSpec((ROW_BLK, 1), lambda i: (i, 0)),
        ],
        out_shape=[
            jax.ShapeDtypeStruct((N, D_HID), jnp.float32),
            jax.ShapeDtypeStruct((N, 1), jnp.float32),
            jax.ShapeDtypeStruct((N, 1), jnp.float32),
        ],
    )(hist, y1)


def _tc2_body(part_ref, ddst_ref, dsrc_ref, b1_ref, w2_ref, q2_ref):
    p = part_ref[0] + part_ref[1]
    h = jnp.maximum(p * ddst_ref[...] + b1_ref[...], 0.0)
    q2_ref[...] = jnp.dot(h * dsrc_ref[...], w2_ref[...],
                          preferred_element_type=jnp.float32)


def _tc2(parts, dinv_dst, dinv_src, b1, w2):
    grid = (N // ROW_BLK,)
    return pl.pallas_call(
        _tc2_body,
        grid=grid,
        in_specs=[
            pl.BlockSpec((NC, ROW_BLK, D_HID), lambda i: (0, i, 0)),
            pl.BlockSpec((ROW_BLK, 1), lambda i: (i, 0)),
            pl.BlockSpec((ROW_BLK, 1), lambda i: (i, 0)),
            pl.BlockSpec((1, D_HID), lambda i: (0, 0)),
            pl.BlockSpec((D_HID, N_CLASS), lambda i: (0, 0)),
        ],
        out_specs=pl.BlockSpec((ROW_BLK, N_CLASS), lambda i: (i, 0)),
        out_shape=jax.ShapeDtypeStruct((N, N_CLASS), jnp.float32),
    )(parts, dinv_dst, dinv_src, b1, w2)


def _tc3_body(part_ref, ddst_ref, b2_ref, out_ref):
    p = part_ref[0] + part_ref[1]
    out_ref[...] = p * ddst_ref[...] + b2_ref[...]


def _tc3(parts, dinv_dst, b2):
    grid = (N // ROW_BLK,)
    return pl.pallas_call(
        _tc3_body,
        grid=grid,
        in_specs=[
            pl.BlockSpec((NC, ROW_BLK, N_CLASS), lambda i: (0, i, 0)),
            pl.BlockSpec((ROW_BLK, 1), lambda i: (i, 0)),
            pl.BlockSpec((1, N_CLASS), lambda i: (0, 0)),
        ],
        out_specs=pl.BlockSpec((ROW_BLK, N_CLASS), lambda i: (i, 0)),
        out_shape=jax.ShapeDtypeStruct((N, N_CLASS), jnp.float32),
    )(parts, dinv_dst, b2)


def kernel(x, edge_index, W1, b1, W2, b2):
    src = edge_index[0]
    dst = edge_index[1]
    src3 = src.reshape(NW, NCHUNK, CHUNK)
    dst3 = dst.reshape(NW, NCHUNK, CHUNK)

    hist = _degrees(src3, dst3)
    y1 = _tc0(x, W1)

    q1, dinv_src, dinv_dst = _tc1(hist, y1)
    parts1 = _message_pass(q1, src3, dst3, D_HID)
    q2 = _tc2(parts1, dinv_dst, dinv_src, b1.reshape(1, D_HID), W2)
    parts2 = _message_pass(q2, src3, dst3, N_CLASS)
    out = _tc3(parts2, dinv_dst, b2.reshape(1, N_CLASS))
    return out
